# Initial kernel scaffold; baseline (speedup 1.0000x reference)
import functools

import jax
import jax.numpy as jnp
from jax import lax
from jax.experimental import pallas as pl
from jax.experimental.pallas import tpu as pltpu
from jax.experimental.pallas import tpu_sc as plsc

_N = 10000
_NPAD = 10240
_E = 320000
_DIN = 128
_HID = 16
_DOUT = 64

_NC = 2
_NS = 16
_NW = _NC * _NS
_CHUNK = 128
_CPW = 79
_EPAD = _NW * _CPW * _CHUNK
_RPW = _NPAD // _NS

_BLK = 2048
_GRID = _NPAD // _BLK


def _make_deg():
  mesh = plsc.VectorSubcoreMesh(core_axis_name="c", subcore_axis_name="s")

  @functools.partial(
      pl.kernel,
      out_type=jax.ShapeDtypeStruct((_NC, _NPAD), jnp.float32),
      mesh=mesh,
      scratch_types=[
          pltpu.VMEM((_CPW, _CHUNK), jnp.int32),
          pltpu.VMEM((_CHUNK,), jnp.float32),
          pltpu.VMEM((_RPW,), jnp.float32),
          pltpu.VMEM_SHARED((_NPAD,), jnp.float32),
      ],
  )
  def deg_kernel(dst_hbm, out_hbm, didx, ones_v, zbuf, acc):
    cid = lax.axis_index("c")
    sid = lax.axis_index("s")
    wid = sid * _NC + cid
    one = jnp.ones((16,), jnp.float32)
    zro = jnp.zeros((16,), jnp.float32)
    for j in range(_CHUNK // 16):
      ones_v[pl.ds(16 * j, 16)] = one

    def zb(i, carry):
      zbuf[pl.ds(i * 16, 16)] = zro
      return carry

    lax.fori_loop(0, _RPW // 16, zb, 0)
    pltpu.sync_copy(zbuf, acc.at[pl.ds(sid * _RPW, _RPW)])
    pltpu.sync_copy(dst_hbm.at[pl.ds(wid * _CPW, _CPW)], didx)
    plsc.subcore_barrier()

    def body(i, carry):
      pltpu.sync_copy(ones_v, acc.at[didx.at[i]], add=True)
      return carry

    lax.fori_loop(0, _CPW, body, 0)
    plsc.subcore_barrier()
    pltpu.sync_copy(acc.at[pl.ds(sid * _RPW, _RPW)],
                    out_hbm.at[cid, pl.ds(sid * _RPW, _RPW)])

  return deg_kernel


def _make_prop(c_dim):
  mesh = plsc.VectorSubcoreMesh(core_axis_name="c", subcore_axis_name="s")

  @functools.partial(
      pl.kernel,
      out_type=jax.ShapeDtypeStruct((_NC, _NPAD, c_dim), jnp.float32),
      mesh=mesh,
      scratch_types=[
          pltpu.VMEM((_CPW, _CHUNK), jnp.int32),
          pltpu.VMEM((_CPW, _CHUNK), jnp.int32),
          pltpu.VMEM((_CHUNK, c_dim), jnp.float32),
          pltpu.VMEM((_RPW, c_dim), jnp.float32),
          pltpu.VMEM_SHARED((_NPAD, c_dim), jnp.float32),
          pltpu.SemaphoreType.DMA,
      ],
  )
  def prop_kernel(src_hbm, dst_hbm, hs_hbm, out_hbm,
                  sidx, didx, rows, zbuf, acc, sem):
    cid = lax.axis_index("c")
    sid = lax.axis_index("s")
    wid = sid * _NC + cid
    zro = jnp.zeros((16,), jnp.float32)

    def zb(i, carry):
      for j in range(c_dim // 16):
        zbuf[i, pl.ds(16 * j, 16)] = zro
      return carry

    lax.fori_loop(0, _RPW, zb, 0)
    pltpu.sync_copy(zbuf, acc.at[pl.ds(sid * _RPW, _RPW)])
    pltpu.sync_copy(src_hbm.at[pl.ds(wid * _CPW, _CPW)], sidx)
    pltpu.sync_copy(dst_hbm.at[pl.ds(wid * _CPW, _CPW)], didx)
    plsc.subcore_barrier()

    def body(i, carry):
      pltpu.async_copy(hs_hbm.at[sidx.at[i]], rows, sem).wait()
      pltpu.sync_copy(rows, acc.at[didx.at[i]], add=True)
      return carry

    lax.fori_loop(0, _CPW, body, 0)
    plsc.subcore_barrier()
    pltpu.sync_copy(acc.at[pl.ds(sid * _RPW, _RPW)],
                    out_hbm.at[cid, pl.ds(sid * _RPW, _RPW)])

  return prop_kernel


_DEG = _make_deg()
_PROP16 = _make_prop(16)
_PROP32 = _make_prop(32)


def _tc1(x, wcat, degp):
  def body(x_ref, w_ref, degp_ref, u0_ref, hs1_ref, dinv_ref):
    deg = degp_ref[0] + degp_ref[1]
    dinv = jnp.where(deg > 0, lax.rsqrt(jnp.where(deg > 0, deg, 1.0)), 0.0)
    u = jnp.dot(x_ref[...], w_ref[...], preferred_element_type=jnp.float32)
    u0_ref[...] = u[:, :_HID]
    hs1_ref[...] = u[:, _HID:] * dinv[:, None]
    dinv_ref[...] = dinv[:, None]

  return pl.pallas_call(
      body,
      grid=(_GRID,),
      in_specs=[
          pl.BlockSpec((_BLK, _DIN), lambda i: (i, 0)),
          pl.BlockSpec((_DIN, 3 * _HID), lambda i: (0, 0)),
          pl.BlockSpec((_NC, _BLK), lambda i: (0, i)),
      ],
      out_specs=[
          pl.BlockSpec((_BLK, _HID), lambda i: (i, 0)),
          pl.BlockSpec((_BLK, 2 * _HID), lambda i: (i, 0)),
          pl.BlockSpec((_BLK, 1), lambda i: (i, 0)),
      ],
      out_shape=[
          jax.ShapeDtypeStruct((_NPAD, _HID), jnp.float32),
          jax.ShapeDtypeStruct((_NPAD, 2 * _HID), jnp.float32),
          jax.ShapeDtypeStruct((_NPAD, 1), jnp.float32),
      ],
  )(x, wcat, degp)


def _tc2(p1, u0, dinv, b1):
  def body(p_ref, u0_ref, d_ref, b_ref, pre_ref, hs2_ref):
    s = p_ref[0] + p_ref[1]
    d = d_ref[...]
    pre_ref[...] = u0_ref[...] + s[:, :_HID] * d + b_ref[...]
    hs2_ref[...] = s[:, _HID:] * (d * d)

  return pl.pallas_call(
      body,
      grid=(_GRID,),
      in_specs=[
          pl.BlockSpec((_NC, _BLK, 2 * _HID), lambda i: (0, i, 0)),
          pl.BlockSpec((_BLK, _HID), lambda i: (i, 0)),
          pl.BlockSpec((_BLK, 1), lambda i: (i, 0)),
          pl.BlockSpec((1, _HID), lambda i: (0, 0)),
      ],
      out_specs=[
          pl.BlockSpec((_BLK, _HID), lambda i: (i, 0)),
          pl.BlockSpec((_BLK, _HID), lambda i: (i, 0)),
      ],
      out_shape=[
          jax.ShapeDtypeStruct((_NPAD, _HID), jnp.float32),
          jax.ShapeDtypeStruct((_NPAD, _HID), jnp.float32),
      ],
  )(p1, u0, dinv, b1)


def _tc3(pre1, p2, dinv):
  def body(pre_ref, p_ref, d_ref, h_ref, hs3_ref):
    s = p_ref[0] + p_ref[1]
    d = d_ref[...]
    h = jnp.maximum(pre_ref[...] + s * d, 0.0)
    h_ref[...] = h
    hs3_ref[...] = h * d

  return pl.pallas_call(
      body,
      grid=(_GRID,),
      in_specs=[
          pl.BlockSpec((_BLK, _HID), lambda i: (i, 0)),
          pl.BlockSpec((_NC, _BLK, _HID), lambda i: (0, i, 0)),
          pl.BlockSpec((_BLK, 1), lambda i: (i, 0)),
      ],
      out_specs=[
          pl.BlockSpec((_BLK, _HID), lambda i: (i, 0)),
          pl.BlockSpec((_BLK, _HID), lambda i: (i, 0)),
      ],
      out_shape=[
          jax.ShapeDtypeStruct((_NPAD, _HID), jnp.float32),
          jax.ShapeDtypeStruct((_NPAD, _HID), jnp.float32),
      ],
  )(pre1, p2, dinv)


def _tc4(p3, dinv):
  def body(p_ref, d_ref, g1_ref, hs4_ref):
    s = p_ref[0] + p_ref[1]
    d = d_ref[...]
    g1 = s * d
    g1_ref[...] = g1
    hs4_ref[...] = g1 * d

  return pl.pallas_call(
      body,
      grid=(_GRID,),
      in_specs=[
          pl.BlockSpec((_NC, _BLK, _HID), lambda i: (0, i, 0)),
          pl.BlockSpec((_BLK, 1), lambda i: (i, 0)),
      ],
      out_specs=[
          pl.BlockSpec((_BLK, _HID), lambda i: (i, 0)),
          pl.BlockSpec((_BLK, _HID), lambda i: (i, 0)),
      ],
      out_shape=[
          jax.ShapeDtypeStruct((_NPAD, _HID), jnp.float32),
          jax.ShapeDtypeStruct((_NPAD, _HID), jnp.float32),
      ],
  )(p3, dinv)


def _tc5(h, g1, p4, dinv, w0, w1, w2, b2):
  def body(h_ref, g1_ref, p_ref, d_ref, w0_ref, w1_ref, w2_ref, b_ref,
           out_ref):
    g2 = (p_ref[0] + p_ref[1]) * d_ref[...]
    out_ref[...] = (
        jnp.dot(h_ref[...], w0_ref[...], preferred_element_type=jnp.float32)
        + jnp.dot(g1_ref[...], w1_ref[...], preferred_element_type=jnp.float32)
        + jnp.dot(g2, w2_ref[...], preferred_element_type=jnp.float32)
        + b_ref[...])

  return pl.pallas_call(
      body,
      grid=(_GRID,),
      in_specs=[
          pl.BlockSpec((_BLK, _HID), lambda i: (i, 0)),
          pl.BlockSpec((_BLK, _HID), lambda i: (i, 0)),
          pl.BlockSpec((_NC, _BLK, _HID), lambda i: (0, i, 0)),
          pl.BlockSpec((_BLK, 1), lambda i: (i, 0)),
          pl.BlockSpec((_HID, _DOUT), lambda i: (0, 0)),
          pl.BlockSpec((_HID, _DOUT), lambda i: (0, 0)),
          pl.BlockSpec((_HID, _DOUT), lambda i: (0, 0)),
          pl.BlockSpec((1, _DOUT), lambda i: (0, 0)),
      ],
      out_specs=pl.BlockSpec((_BLK, _DOUT), lambda i: (i, 0)),
      out_shape=jax.ShapeDtypeStruct((_NPAD, _DOUT), jnp.float32),
  )(h, g1, p4, dinv, w0, w1, w2, b2)


def kernel(x, edge_index, W1_0, W1_1, W1_2, b1, W2_0, W2_1, W2_2, b2):
  src = edge_index[0]
  dst = edge_index[1]
  pad_e = _EPAD - _E
  pad_idx = jnp.full((pad_e,), _N, jnp.int32)
  src_p = jnp.concatenate([src, pad_idx]).reshape(_NW * _CPW, _CHUNK)
  dst_p = jnp.concatenate([dst, pad_idx]).reshape(_NW * _CPW, _CHUNK)
  x_p = jnp.pad(x, ((0, _NPAD - _N), (0, 0)))
  w1cat = jnp.concatenate([W1_0, W1_1, W1_2], axis=1)

  degp = _DEG(dst_p)
  u0, hs1, dinv = _tc1(x_p, w1cat, degp)
  p1 = _PROP32(src_p, dst_p, hs1)
  pre1, hs2 = _tc2(p1, u0, dinv, b1.reshape(1, _HID))
  p2 = _PROP16(src_p, dst_p, hs2)
  h, hs3 = _tc3(pre1, p2, dinv)
  p3 = _PROP16(src_p, dst_p, hs3)
  g1, hs4 = _tc4(p3, dinv)
  p4 = _PROP16(src_p, dst_p, hs4)
  out = _tc5(h, g1, p4, dinv, W2_0, W2_1, W2_2, b2.reshape(1, _DOUT))
  return out[:_N]


# speedup vs baseline: 24.7796x; 24.7796x over previous
import functools

import jax
import jax.numpy as jnp
from jax import lax
from jax.experimental import pallas as pl
from jax.experimental.pallas import tpu as pltpu
from jax.experimental.pallas import tpu_sc as plsc

_N = 10000
_NPAD = 10240
_E = 320000
_DIN = 128
_HID = 16
_DOUT = 64

_NC = 2
_NS = 16
_NW = _NC * _NS
_KRCH = 512
_NIDX = 20
_EPAD = _NW * _NIDX * _KRCH
_RPW = _NPAD // _NS
_NBUF = 4
_ZROWS = 80

_BLK = 2048
_GRID = _NPAD // _BLK


def _make_deg():
  mesh = plsc.VectorSubcoreMesh(core_axis_name="c", subcore_axis_name="s")

  @functools.partial(
      pl.kernel,
      out_type=jax.ShapeDtypeStruct((_NC * _NPAD,), jnp.float32),
      mesh=mesh,
      scratch_types=[
          pltpu.VMEM((_NIDX, _KRCH), jnp.int32),
          pltpu.VMEM((_KRCH,), jnp.float32),
          pltpu.VMEM((_RPW,), jnp.float32),
          pltpu.VMEM_SHARED((_NPAD,), jnp.float32),
          pltpu.SemaphoreType.DMA,
          pltpu.SemaphoreType.DMA,
      ],
      compiler_params=pltpu.CompilerParams(use_tc_tiling_on_sc=False),
  )
  def deg_kernel(dst_hbm, out_hbm, didx, ones_v, zbuf, acc, sem0, sem1):
    cid = lax.axis_index("c")
    sid = lax.axis_index("s")
    wid = sid * _NC + cid
    one = jnp.ones((16,), jnp.float32)
    zro = jnp.zeros((16,), jnp.float32)
    for j in range(_KRCH // 16):
      ones_v[pl.ds(16 * j, 16)] = one

    def zb(i, carry):
      zbuf[pl.ds(i * 16, 16)] = zro
      return carry

    lax.fori_loop(0, _RPW // 16, zb, 0)
    pltpu.sync_copy(zbuf, acc.at[pl.ds(sid * _RPW, _RPW)])
    pltpu.sync_copy(dst_hbm.at[pl.ds(wid * _NIDX, _NIDX)], didx)
    plsc.subcore_barrier()
    sems = (sem0, sem1)
    descs = []
    for i in range(_NIDX):
      descs.append(
          pltpu.async_copy(ones_v, acc.at[didx.at[i]], sems[i % 2], add=True))
    for d in descs:
      d.wait()
    plsc.subcore_barrier()
    pltpu.sync_copy(acc.at[pl.ds(sid * _RPW, _RPW)],
                    out_hbm.at[pl.ds(cid * _NPAD + sid * _RPW, _RPW)])

  return deg_kernel


def _make_prop(c_dim):
  mesh = plsc.VectorSubcoreMesh(core_axis_name="c", subcore_axis_name="s")

  @functools.partial(
      pl.kernel,
      out_type=jax.ShapeDtypeStruct((_NC, _NPAD, c_dim), jnp.float32),
      mesh=mesh,
      scratch_types=(
          [pltpu.VMEM((_NIDX, _KRCH), jnp.int32),
           pltpu.VMEM((_NIDX, _KRCH), jnp.int32)]
          + [pltpu.VMEM((_KRCH, c_dim), jnp.float32) for _ in range(_NBUF)]
          + [pltpu.VMEM((_ZROWS, c_dim), jnp.float32),
             pltpu.VMEM_SHARED((_NPAD, c_dim), jnp.float32)]
          + [pltpu.SemaphoreType.DMA for _ in range(2 * _NBUF)]
      ),
      compiler_params=pltpu.CompilerParams(use_tc_tiling_on_sc=False),
  )
  def prop_kernel(src_hbm, dst_hbm, hs_hbm, out_hbm, *refs):
    sidx, didx = refs[0], refs[1]
    rows = refs[2:2 + _NBUF]
    zbuf = refs[2 + _NBUF]
    acc = refs[3 + _NBUF]
    gsems = refs[4 + _NBUF:4 + 2 * _NBUF]
    ssems = refs[4 + 2 * _NBUF:4 + 3 * _NBUF]
    cid = lax.axis_index("c")
    sid = lax.axis_index("s")
    wid = sid * _NC + cid
    zro = jnp.zeros((16,), jnp.float32)

    def zb(i, carry):
      for j in range(c_dim // 16):
        zbuf[i, pl.ds(16 * j, 16)] = zro
      return carry

    lax.fori_loop(0, _ZROWS, zb, 0)
    for k in range(_RPW // _ZROWS):
      pltpu.sync_copy(zbuf, acc.at[pl.ds(sid * _RPW + k * _ZROWS, _ZROWS)])
    pltpu.sync_copy(src_hbm.at[pl.ds(wid * _NIDX, _NIDX)], sidx)
    pltpu.sync_copy(dst_hbm.at[pl.ds(wid * _NIDX, _NIDX)], didx)
    plsc.subcore_barrier()

    g = [None] * _NIDX
    s = [None] * _NIDX
    for i in range(_NBUF - 1):
      g[i] = pltpu.async_copy(hs_hbm.at[sidx.at[i]], rows[i % _NBUF],
                              gsems[i % _NBUF])
    for i in range(_NIDX):
      b = i % _NBUF
      g[i].wait()
      s[i] = pltpu.async_copy(rows[b], acc.at[didx.at[i]], ssems[b], add=True)
      j = i + _NBUF - 1
      if j < _NIDX:
        bj = j % _NBUF
        if i >= 1:
          s[i - 1].wait()
        g[j] = pltpu.async_copy(hs_hbm.at[sidx.at[j]], rows[bj], gsems[bj])
    for i in range(max(0, _NIDX - _NBUF), _NIDX):
      if s[i] is not None:
        s[i].wait()
    plsc.subcore_barrier()
    pltpu.sync_copy(acc.at[pl.ds(sid * _RPW, _RPW)],
                    out_hbm.at[cid, pl.ds(sid * _RPW, _RPW)])

  return prop_kernel


_DEG = _make_deg()
_PROP16 = _make_prop(16)
_PROP32 = _make_prop(32)


def _tc0(x, wcat):
  def body(x_ref, w_ref, u_ref):
    u_ref[...] = jnp.dot(x_ref[...], w_ref[...],
                         preferred_element_type=jnp.float32)

  return pl.pallas_call(
      body,
      grid=(_GRID,),
      in_specs=[
          pl.BlockSpec((_BLK, _DIN), lambda i: (i, 0)),
          pl.BlockSpec((_DIN, 3 * _HID), lambda i: (0, 0)),
      ],
      out_specs=pl.BlockSpec((_BLK, 3 * _HID), lambda i: (i, 0)),
      out_shape=jax.ShapeDtypeStruct((_NPAD, 3 * _HID), jnp.float32),
  )(x, wcat)


def _tc1(u, degp):
  def body(u_ref, degp_ref, u0_ref, hs1_ref, dinv_ref):
    deg = degp_ref[0] + degp_ref[1]
    dinv = jnp.where(deg > 0, lax.rsqrt(jnp.where(deg > 0, deg, 1.0)), 0.0)
    u = u_ref[...]
    u0_ref[...] = u[:, :_HID]
    hs1_ref[...] = u[:, _HID:] * dinv[:, None]
    dinv_ref[...] = dinv[:, None]

  return pl.pallas_call(
      body,
      grid=(_GRID,),
      in_specs=[
          pl.BlockSpec((_BLK, 3 * _HID), lambda i: (i, 0)),
          pl.BlockSpec((_NC, _BLK), lambda i: (0, i)),
      ],
      out_specs=[
          pl.BlockSpec((_BLK, _HID), lambda i: (i, 0)),
          pl.BlockSpec((_BLK, 2 * _HID), lambda i: (i, 0)),
          pl.BlockSpec((_BLK, 1), lambda i: (i, 0)),
      ],
      out_shape=[
          jax.ShapeDtypeStruct((_NPAD, _HID), jnp.float32),
          jax.ShapeDtypeStruct((_NPAD, 2 * _HID), jnp.float32),
          jax.ShapeDtypeStruct((_NPAD, 1), jnp.float32),
      ],
  )(u, degp)


def _tc2(p1, u0, dinv, b1):
  def body(p_ref, u0_ref, d_ref, b_ref, pre_ref, hs2_ref):
    s = p_ref[0] + p_ref[1]
    d = d_ref[...]
    pre_ref[...] = u0_ref[...] + s[:, :_HID] * d + b_ref[...]
    hs2_ref[...] = s[:, _HID:] * (d * d)

  return pl.pallas_call(
      body,
      grid=(_GRID,),
      in_specs=[
          pl.BlockSpec((_NC, _BLK, 2 * _HID), lambda i: (0, i, 0)),
          pl.BlockSpec((_BLK, _HID), lambda i: (i, 0)),
          pl.BlockSpec((_BLK, 1), lambda i: (i, 0)),
          pl.BlockSpec((1, _HID), lambda i: (0, 0)),
      ],
      out_specs=[
          pl.BlockSpec((_BLK, _HID), lambda i: (i, 0)),
          pl.BlockSpec((_BLK, _HID), lambda i: (i, 0)),
      ],
      out_shape=[
          jax.ShapeDtypeStruct((_NPAD, _HID), jnp.float32),
          jax.ShapeDtypeStruct((_NPAD, _HID), jnp.float32),
      ],
  )(p1, u0, dinv, b1)


def _tc3(pre1, p2, dinv):
  def body(pre_ref, p_ref, d_ref, h_ref, hs3_ref):
    s = p_ref[0] + p_ref[1]
    d = d_ref[...]
    h = jnp.maximum(pre_ref[...] + s * d, 0.0)
    h_ref[...] = h
    hs3_ref[...] = h * d

  return pl.pallas_call(
      body,
      grid=(_GRID,),
      in_specs=[
          pl.BlockSpec((_BLK, _HID), lambda i: (i, 0)),
          pl.BlockSpec((_NC, _BLK, _HID), lambda i: (0, i, 0)),
          pl.BlockSpec((_BLK, 1), lambda i: (i, 0)),
      ],
      out_specs=[
          pl.BlockSpec((_BLK, _HID), lambda i: (i, 0)),
          pl.BlockSpec((_BLK, _HID), lambda i: (i, 0)),
      ],
      out_shape=[
          jax.ShapeDtypeStruct((_NPAD, _HID), jnp.float32),
          jax.ShapeDtypeStruct((_NPAD, _HID), jnp.float32),
      ],
  )(pre1, p2, dinv)


def _tc4(p3, dinv):
  def body(p_ref, d_ref, g1_ref, hs4_ref):
    s = p_ref[0] + p_ref[1]
    d = d_ref[...]
    g1 = s * d
    g1_ref[...] = g1
    hs4_ref[...] = g1 * d

  return pl.pallas_call(
      body,
      grid=(_GRID,),
      in_specs=[
          pl.BlockSpec((_NC, _BLK, _HID), lambda i: (0, i, 0)),
          pl.BlockSpec((_BLK, 1), lambda i: (i, 0)),
      ],
      out_specs=[
          pl.BlockSpec((_BLK, _HID), lambda i: (i, 0)),
          pl.BlockSpec((_BLK, _HID), lambda i: (i, 0)),
      ],
      out_shape=[
          jax.ShapeDtypeStruct((_NPAD, _HID), jnp.float32),
          jax.ShapeDtypeStruct((_NPAD, _HID), jnp.float32),
      ],
  )(p3, dinv)


def _tc5a(h, g1, w0, w1, b2):
  def body(h_ref, g1_ref, w0_ref, w1_ref, b_ref, out_ref):
    out_ref[...] = (
        jnp.dot(h_ref[...], w0_ref[...], preferred_element_type=jnp.float32)
        + jnp.dot(g1_ref[...], w1_ref[...], preferred_element_type=jnp.float32)
        + b_ref[...])

  return pl.pallas_call(
      body,
      grid=(_GRID,),
      in_specs=[
          pl.BlockSpec((_BLK, _HID), lambda i: (i, 0)),
          pl.BlockSpec((_BLK, _HID), lambda i: (i, 0)),
          pl.BlockSpec((_HID, _DOUT), lambda i: (0, 0)),
          pl.BlockSpec((_HID, _DOUT), lambda i: (0, 0)),
          pl.BlockSpec((1, _DOUT), lambda i: (0, 0)),
      ],
      out_specs=pl.BlockSpec((_BLK, _DOUT), lambda i: (i, 0)),
      out_shape=jax.ShapeDtypeStruct((_NPAD, _DOUT), jnp.float32),
  )(h, g1, w0, w1, b2)


def _tc5b(tmp, p4, dinv, w2):
  def body(tmp_ref, p_ref, d_ref, w2_ref, out_ref):
    g2 = (p_ref[0] + p_ref[1]) * d_ref[...]
    out_ref[...] = tmp_ref[...] + jnp.dot(
        g2, w2_ref[...], preferred_element_type=jnp.float32)

  return pl.pallas_call(
      body,
      grid=(_GRID,),
      in_specs=[
          pl.BlockSpec((_BLK, _DOUT), lambda i: (i, 0)),
          pl.BlockSpec((_NC, _BLK, _HID), lambda i: (0, i, 0)),
          pl.BlockSpec((_BLK, 1), lambda i: (i, 0)),
          pl.BlockSpec((_HID, _DOUT), lambda i: (0, 0)),
      ],
      out_specs=pl.BlockSpec((_BLK, _DOUT), lambda i: (i, 0)),
      out_shape=jax.ShapeDtypeStruct((_NPAD, _DOUT), jnp.float32),
  )(tmp, p4, dinv, w2)


def kernel(x, edge_index, W1_0, W1_1, W1_2, b1, W2_0, W2_1, W2_2, b2):
  src = edge_index[0]
  dst = edge_index[1]
  pad_e = _EPAD - _E
  pad_idx = jnp.full((pad_e,), _N, jnp.int32)
  src_p = jnp.concatenate([src, pad_idx]).reshape(_NW * _NIDX, _KRCH)
  dst_p = jnp.concatenate([dst, pad_idx]).reshape(_NW * _NIDX, _KRCH)
  x_p = jnp.pad(x, ((0, _NPAD - _N), (0, 0)))
  w1cat = jnp.concatenate([W1_0, W1_1, W1_2], axis=1)

  degp = _DEG(dst_p).reshape(_NC, _NPAD)
  u = _tc0(x_p, w1cat)
  u0, hs1, dinv = _tc1(u, degp)
  p1 = _PROP32(src_p, dst_p, hs1)
  pre1, hs2 = _tc2(p1, u0, dinv, b1.reshape(1, _HID))
  p2 = _PROP16(src_p, dst_p, hs2)
  h, hs3 = _tc3(pre1, p2, dinv)
  p3 = _PROP16(src_p, dst_p, hs3)
  g1, hs4 = _tc4(p3, dinv)
  tmp = _tc5a(h, g1, W2_0, W2_1, b2.reshape(1, _DOUT))
  p4 = _PROP16(src_p, dst_p, hs4)
  out = _tc5b(tmp, p4, dinv, W2_2)
  return out[:_N]


# speedup vs baseline: 26.7388x; 1.0791x over previous
import functools

import jax
import jax.numpy as jnp
from jax import lax
from jax.experimental import pallas as pl
from jax.experimental.pallas import tpu as pltpu
from jax.experimental.pallas import tpu_sc as plsc

_N = 10000
_NPAD = 10240
_E = 320000
_DIN = 128
_HID = 16
_DOUT = 64

_NC = 2
_NS = 16
_NW = _NC * _NS
_KRCH = 512
_NIDX = 20
_EPAD = _NW * _NIDX * _KRCH
_RPW = _NPAD // _NS
_NBUF = 4
_ZROWS = 80

_BLK = 2048
_GRID = _NPAD // _BLK


def _make_deg():
  mesh = plsc.VectorSubcoreMesh(core_axis_name="c", subcore_axis_name="s")

  @functools.partial(
      pl.kernel,
      out_type=jax.ShapeDtypeStruct((_NC * _NPAD,), jnp.float32),
      mesh=mesh,
      scratch_types=[
          pltpu.VMEM((_NIDX, _KRCH), jnp.int32),
          pltpu.VMEM((_KRCH,), jnp.float32),
          pltpu.VMEM((_RPW,), jnp.float32),
          pltpu.VMEM_SHARED((_NPAD,), jnp.float32),
          pltpu.SemaphoreType.DMA,
          pltpu.SemaphoreType.DMA,
      ],
      compiler_params=pltpu.CompilerParams(use_tc_tiling_on_sc=False),
  )
  def deg_kernel(dst_hbm, out_hbm, didx, ones_v, zbuf, acc, sem0, sem1):
    cid = lax.axis_index("c")
    sid = lax.axis_index("s")
    wid = sid * _NC + cid
    one = jnp.ones((16,), jnp.float32)
    zro = jnp.zeros((16,), jnp.float32)
    for j in range(_KRCH // 16):
      ones_v[pl.ds(16 * j, 16)] = one

    def zb(i, carry):
      zbuf[pl.ds(i * 16, 16)] = zro
      return carry

    lax.fori_loop(0, _RPW // 16, zb, 0)
    pltpu.sync_copy(zbuf, acc.at[pl.ds(sid * _RPW, _RPW)])
    pltpu.sync_copy(dst_hbm.at[pl.ds(wid * _NIDX, _NIDX)], didx)
    plsc.subcore_barrier()
    sems = (sem0, sem1)
    descs = []
    for i in range(_NIDX):
      descs.append(
          pltpu.async_copy(ones_v, acc.at[didx.at[i]], sems[i % 2], add=True))
    for d in descs:
      d.wait()
    plsc.subcore_barrier()
    pltpu.sync_copy(acc.at[pl.ds(sid * _RPW, _RPW)],
                    out_hbm.at[pl.ds(cid * _NPAD + sid * _RPW, _RPW)])

  return deg_kernel


_CHA = 28
_CHB = 12
_CHMAX = max(_CHA, _CHB)
_EROWS = 16 * (_CHA + _CHB) + (_CHMAX - min(_CHA, _CHB))


def _make_prop(c_dim):
  mesh = plsc.VectorSubcoreMesh(core_axis_name="c", subcore_axis_name="s")

  @functools.partial(
      pl.kernel,
      out_type=jax.ShapeDtypeStruct((_NC, _NPAD, c_dim), jnp.float32),
      mesh=mesh,
      scratch_types=(
          [pltpu.VMEM((_CHMAX, _KRCH), jnp.int32),
           pltpu.VMEM((_CHMAX, _KRCH), jnp.int32)]
          + [pltpu.VMEM((_KRCH, c_dim), jnp.float32) for _ in range(_NBUF)]
          + [pltpu.VMEM((_ZROWS, c_dim), jnp.float32),
             pltpu.VMEM_SHARED((_NPAD, c_dim), jnp.float32)]
          + [pltpu.SemaphoreType.DMA for _ in range(2 * _NBUF)]
      ),
      compiler_params=pltpu.CompilerParams(use_tc_tiling_on_sc=False),
  )
  def prop_kernel(src_hbm, dst_hbm, hs_hbm, out_hbm, *refs):
    sidx, didx = refs[0], refs[1]
    rows = refs[2:2 + _NBUF]
    zbuf = refs[2 + _NBUF]
    acc = refs[3 + _NBUF]
    gsems = refs[4 + _NBUF:4 + 2 * _NBUF]
    ssems = refs[4 + 2 * _NBUF:4 + 3 * _NBUF]
    cid = lax.axis_index("c")
    sid = lax.axis_index("s")
    cnt = jnp.where(cid == 0, _CHA, _CHB)
    base = jnp.where(cid == 0, sid * _CHA, 16 * _CHA + sid * _CHB)
    zro = jnp.zeros((16,), jnp.float32)

    def zb(i, carry):
      for j in range(c_dim // 16):
        zbuf[i, pl.ds(16 * j, 16)] = zro
      return carry

    lax.fori_loop(0, _ZROWS, zb, 0)
    for k in range(_RPW // _ZROWS):
      pltpu.sync_copy(zbuf, acc.at[pl.ds(sid * _RPW + k * _ZROWS, _ZROWS)])
    pltpu.sync_copy(src_hbm.at[pl.ds(base, _CHMAX)], sidx)
    pltpu.sync_copy(dst_hbm.at[pl.ds(base, _CHMAX)], didx)
    plsc.subcore_barrier()

    def gath(i, b):
      return pltpu.async_copy(hs_hbm.at[sidx.at[i]], rows[b], gsems[b])

    def scat(i, b):
      return pltpu.async_copy(rows[b], acc.at[didx.at[i]], ssems[b], add=True)

    for b in range(_NBUF - 1):
      gath(b, b)

    def group(g, carry):
      for b in range(_NBUF):
        i = g * _NBUF + b
        pltpu.make_async_copy(hs_hbm.at[sidx.at[i]], rows[b], gsems[b]).wait()
        scat(i, b)
        bp = (b - 1) % _NBUF

        @pl.when(i + _NBUF - 1 < cnt)
        def _():
          @pl.when(i >= 1)
          def _():
            pltpu.make_async_copy(rows[bp], acc.at[didx.at[i - 1]],
                                  ssems[bp]).wait()
          gath(i + _NBUF - 1, bp)

      return carry

    lax.fori_loop(0, cnt // _NBUF, group, 0)
    for k in range(_NBUF):
      i = cnt - _NBUF + k
      b = k
      pltpu.make_async_copy(rows[b], acc.at[didx.at[i]], ssems[b]).wait()
    plsc.subcore_barrier()
    pltpu.sync_copy(acc.at[pl.ds(sid * _RPW, _RPW)],
                    out_hbm.at[cid, pl.ds(sid * _RPW, _RPW)])

  return prop_kernel


_DEG = _make_deg()
_PROP16 = _make_prop(16)
_PROP32 = _make_prop(32)


def _tc0(x, wcat):
  def body(x_ref, w_ref, u_ref):
    u_ref[...] = jnp.dot(x_ref[...], w_ref[...],
                         preferred_element_type=jnp.float32)

  return pl.pallas_call(
      body,
      grid=(_GRID,),
      in_specs=[
          pl.BlockSpec((_BLK, _DIN), lambda i: (i, 0)),
          pl.BlockSpec((_DIN, 3 * _HID), lambda i: (0, 0)),
      ],
      out_specs=pl.BlockSpec((_BLK, 3 * _HID), lambda i: (i, 0)),
      out_shape=jax.ShapeDtypeStruct((_NPAD, 3 * _HID), jnp.float32),
  )(x, wcat)


def _tc1(u, degp):
  def body(u_ref, degp_ref, u0_ref, hs1_ref, dinv_ref):
    deg = degp_ref[0] + degp_ref[1]
    dinv = jnp.where(deg > 0, lax.rsqrt(jnp.where(deg > 0, deg, 1.0)), 0.0)
    u = u_ref[...]
    u0_ref[...] = u[:, :_HID]
    hs1_ref[...] = u[:, _HID:] * dinv[:, None]
    dinv_ref[...] = dinv[:, None]

  return pl.pallas_call(
      body,
      grid=(_GRID,),
      in_specs=[
          pl.BlockSpec((_BLK, 3 * _HID), lambda i: (i, 0)),
          pl.BlockSpec((_NC, _BLK), lambda i: (0, i)),
      ],
      out_specs=[
          pl.BlockSpec((_BLK, _HID), lambda i: (i, 0)),
          pl.BlockSpec((_BLK, 2 * _HID), lambda i: (i, 0)),
          pl.BlockSpec((_BLK, 1), lambda i: (i, 0)),
      ],
      out_shape=[
          jax.ShapeDtypeStruct((_NPAD, _HID), jnp.float32),
          jax.ShapeDtypeStruct((_NPAD, 2 * _HID), jnp.float32),
          jax.ShapeDtypeStruct((_NPAD, 1), jnp.float32),
      ],
  )(u, degp)


def _tc2(p1, u0, dinv, b1):
  def body(p_ref, u0_ref, d_ref, b_ref, pre_ref, hs2_ref):
    s = p_ref[0] + p_ref[1]
    d = d_ref[...]
    pre_ref[...] = u0_ref[...] + s[:, :_HID] * d + b_ref[...]
    hs2_ref[...] = s[:, _HID:] * (d * d)

  return pl.pallas_call(
      body,
      grid=(_GRID,),
      in_specs=[
          pl.BlockSpec((_NC, _BLK, 2 * _HID), lambda i: (0, i, 0)),
          pl.BlockSpec((_BLK, _HID), lambda i: (i, 0)),
          pl.BlockSpec((_BLK, 1), lambda i: (i, 0)),
          pl.BlockSpec((1, _HID), lambda i: (0, 0)),
      ],
      out_specs=[
          pl.BlockSpec((_BLK, _HID), lambda i: (i, 0)),
          pl.BlockSpec((_BLK, _HID), lambda i: (i, 0)),
      ],
      out_shape=[
          jax.ShapeDtypeStruct((_NPAD, _HID), jnp.float32),
          jax.ShapeDtypeStruct((_NPAD, _HID), jnp.float32),
      ],
  )(p1, u0, dinv, b1)


def _tc3(pre1, p2, dinv):
  def body(pre_ref, p_ref, d_ref, h_ref, hs3_ref):
    s = p_ref[0] + p_ref[1]
    d = d_ref[...]
    h = jnp.maximum(pre_ref[...] + s * d, 0.0)
    h_ref[...] = h
    hs3_ref[...] = h * d

  return pl.pallas_call(
      body,
      grid=(_GRID,),
      in_specs=[
          pl.BlockSpec((_BLK, _HID), lambda i: (i, 0)),
          pl.BlockSpec((_NC, _BLK, _HID), lambda i: (0, i, 0)),
          pl.BlockSpec((_BLK, 1), lambda i: (i, 0)),
      ],
      out_specs=[
          pl.BlockSpec((_BLK, _HID), lambda i: (i, 0)),
          pl.BlockSpec((_BLK, _HID), lambda i: (i, 0)),
      ],
      out_shape=[
          jax.ShapeDtypeStruct((_NPAD, _HID), jnp.float32),
          jax.ShapeDtypeStruct((_NPAD, _HID), jnp.float32),
      ],
  )(pre1, p2, dinv)


def _tc4(p3, dinv):
  def body(p_ref, d_ref, g1_ref, hs4_ref):
    s = p_ref[0] + p_ref[1]
    d = d_ref[...]
    g1 = s * d
    g1_ref[...] = g1
    hs4_ref[...] = g1 * d

  return pl.pallas_call(
      body,
      grid=(_GRID,),
      in_specs=[
          pl.BlockSpec((_NC, _BLK, _HID), lambda i: (0, i, 0)),
          pl.BlockSpec((_BLK, 1), lambda i: (i, 0)),
      ],
      out_specs=[
          pl.BlockSpec((_BLK, _HID), lambda i: (i, 0)),
          pl.BlockSpec((_BLK, _HID), lambda i: (i, 0)),
      ],
      out_shape=[
          jax.ShapeDtypeStruct((_NPAD, _HID), jnp.float32),
          jax.ShapeDtypeStruct((_NPAD, _HID), jnp.float32),
      ],
  )(p3, dinv)


def _tc5a(h, g1, w0, w1, b2):
  def body(h_ref, g1_ref, w0_ref, w1_ref, b_ref, out_ref):
    out_ref[...] = (
        jnp.dot(h_ref[...], w0_ref[...], preferred_element_type=jnp.float32)
        + jnp.dot(g1_ref[...], w1_ref[...], preferred_element_type=jnp.float32)
        + b_ref[...])

  return pl.pallas_call(
      body,
      grid=(_GRID,),
      in_specs=[
          pl.BlockSpec((_BLK, _HID), lambda i: (i, 0)),
          pl.BlockSpec((_BLK, _HID), lambda i: (i, 0)),
          pl.BlockSpec((_HID, _DOUT), lambda i: (0, 0)),
          pl.BlockSpec((_HID, _DOUT), lambda i: (0, 0)),
          pl.BlockSpec((1, _DOUT), lambda i: (0, 0)),
      ],
      out_specs=pl.BlockSpec((_BLK, _DOUT), lambda i: (i, 0)),
      out_shape=jax.ShapeDtypeStruct((_NPAD, _DOUT), jnp.float32),
  )(h, g1, w0, w1, b2)


def _tc5b(tmp, p4, dinv, w2):
  def body(tmp_ref, p_ref, d_ref, w2_ref, out_ref):
    g2 = (p_ref[0] + p_ref[1]) * d_ref[...]
    out_ref[...] = tmp_ref[...] + jnp.dot(
        g2, w2_ref[...], preferred_element_type=jnp.float32)

  return pl.pallas_call(
      body,
      grid=(_GRID,),
      in_specs=[
          pl.BlockSpec((_BLK, _DOUT), lambda i: (i, 0)),
          pl.BlockSpec((_NC, _BLK, _HID), lambda i: (0, i, 0)),
          pl.BlockSpec((_BLK, 1), lambda i: (i, 0)),
          pl.BlockSpec((_HID, _DOUT), lambda i: (0, 0)),
      ],
      out_specs=pl.BlockSpec((_BLK, _DOUT), lambda i: (i, 0)),
      out_shape=jax.ShapeDtypeStruct((_NPAD, _DOUT), jnp.float32),
  )(tmp, p4, dinv, w2)


def kernel(x, edge_index, W1_0, W1_1, W1_2, b1, W2_0, W2_1, W2_2, b2):
  src = edge_index[0]
  dst = edge_index[1]
  pad_e = _EROWS * _KRCH - _E
  pad_idx = jnp.full((pad_e,), _N, jnp.int32)
  src_p = jnp.concatenate([src, pad_idx]).reshape(_EROWS, _KRCH)
  dst_p = jnp.concatenate([dst, pad_idx]).reshape(_EROWS, _KRCH)
  x_p = jnp.pad(x, ((0, _NPAD - _N), (0, 0)))
  w1cat = jnp.concatenate([W1_0, W1_1, W1_2], axis=1)

  degp = _DEG(dst_p).reshape(_NC, _NPAD)
  u = _tc0(x_p, w1cat)
  u0, hs1, dinv = _tc1(u, degp)
  p1 = _PROP32(src_p, dst_p, hs1)
  pre1, hs2 = _tc2(p1, u0, dinv, b1.reshape(1, _HID))
  p2 = _PROP16(src_p, dst_p, hs2)
  h, hs3 = _tc3(pre1, p2, dinv)
  p3 = _PROP16(src_p, dst_p, hs3)
  g1, hs4 = _tc4(p3, dinv)
  tmp = _tc5a(h, g1, W2_0, W2_1, b2.reshape(1, _DOUT))
  p4 = _PROP16(src_p, dst_p, hs4)
  out = _tc5b(tmp, p4, dinv, W2_2)
  return out[:_N]
